# baseline (device time: 43919 ns/iter reference)
import jax
import jax.numpy as jnp
from jax import lax
from jax.experimental import pallas as pl
from jax.experimental.pallas import tpu as pltpu

N_DEV = 4
N_TOK = 2048
D_MODEL = 512
H = 1024
N_EXP = 16
E_LOC = 4
CAP = 102
M_OUT = N_TOK // N_DEV


def _body(x_ref, w_ref, mask_ref, out_ref, xbf_ref, wbf_ref,
          dest_ref, recv_ref, send_sems, recv_sems):
    my = lax.axis_index("i")

    bar = pltpu.get_barrier_semaphore()
    for k in range(1, N_DEV):
        pl.semaphore_signal(
            bar, inc=1,
            device_id=((my + k) % N_DEV,),
            device_id_type=pl.DeviceIdType.MESH,
        )
    pl.semaphore_wait(bar, N_DEV - 1)

    xbf_ref[...] = x_ref[...].astype(jnp.bfloat16)
    wbf_ref[...] = w_ref[...].astype(jnp.bfloat16)

    def block(t):
        xb = xbf_ref[pl.ds(t * M_OUT, M_OUT), :]
        mb = mask_ref[pl.ds(t * M_OUT, M_OUT), :].astype(jnp.bfloat16)
        acc = jnp.dot(
            xb * mb[:, 0:1], wbf_ref[0], preferred_element_type=jnp.float32
        )
        for le in range(1, E_LOC):
            acc += jnp.dot(
                xb * mb[:, le:le + 1], wbf_ref[le],
                preferred_element_type=jnp.float32,
            )
        return acc

    rdmas = []
    for j in range(1, N_DEV):
        t = (my + j) % N_DEV
        dest_ref[j - 1] = block(t).astype(jnp.bfloat16)
        rdma = pltpu.make_async_remote_copy(
            src_ref=dest_ref.at[j - 1],
            dst_ref=recv_ref.at[3 - j],
            send_sem=send_sems.at[j - 1],
            recv_sem=recv_sems.at[3 - j],
            device_id=(t,),
            device_id_type=pl.DeviceIdType.MESH,
        )
        rdma.start()
        rdmas.append(rdma)

    out_ref[...] = block(my)

    for rdma in rdmas:
        rdma.wait_recv()

    out_ref[...] += (
        recv_ref[0].astype(jnp.float32)
        + recv_ref[1].astype(jnp.float32)
        + recv_ref[2].astype(jnp.float32)
    )

    for rdma in rdmas:
        rdma.wait_send()


def kernel(x, router_W, route_idx, expert_W):
    del router_W
    my = lax.axis_index("i")

    idx = route_idx[:, 0].astype(jnp.int32)
    oh4 = idx[:, None] == (
        my * E_LOC + jnp.arange(E_LOC, dtype=jnp.int32)
    )[None, :]
    cum4 = jnp.cumsum(oh4.astype(jnp.int32), axis=0)
    mask = (oh4 & (cum4 <= CAP)).astype(jnp.float32)

    return pl.pallas_call(
        _body,
        out_shape=jax.ShapeDtypeStruct((M_OUT, H), jnp.float32),
        in_specs=[
            pl.BlockSpec(memory_space=pltpu.VMEM),
            pl.BlockSpec(memory_space=pltpu.VMEM),
            pl.BlockSpec(memory_space=pltpu.VMEM),
        ],
        out_specs=pl.BlockSpec(memory_space=pltpu.VMEM),
        scratch_shapes=[
            pltpu.VMEM((N_TOK, D_MODEL), jnp.bfloat16),
            pltpu.VMEM((E_LOC, D_MODEL, H), jnp.bfloat16),
            pltpu.VMEM((N_DEV - 1, M_OUT, H), jnp.bfloat16),
            pltpu.VMEM((N_DEV - 1, M_OUT, H), jnp.bfloat16),
            pltpu.SemaphoreType.DMA((N_DEV - 1,)),
            pltpu.SemaphoreType.DMA((N_DEV - 1,)),
        ],
        compiler_params=pltpu.CompilerParams(collective_id=0),
    )(x, expert_W, mask)


# device time: 41832 ns/iter; 1.0499x vs baseline; 1.0499x over previous
import jax
import jax.numpy as jnp
from jax import lax
from jax.experimental import pallas as pl
from jax.experimental.pallas import tpu as pltpu

N_DEV = 4
N_TOK = 2048
D_MODEL = 512
H = 1024
N_EXP = 16
E_LOC = 4
CAP = 102
M_OUT = N_TOK // N_DEV


def _body(x_ref, w_ref, ridx_ref, out_ref, xbf_ref, wbf_ref, mask_ref,
          dest_ref, recv_ref, send_sems, recv_sems):
    my = lax.axis_index("i")

    bar = pltpu.get_barrier_semaphore()
    for k in range(1, N_DEV):
        pl.semaphore_signal(
            bar, inc=1,
            device_id=((my + k) % N_DEV,),
            device_id_type=pl.DeviceIdType.MESH,
        )
    pl.semaphore_wait(bar, N_DEV - 1)

    idx = jnp.broadcast_to(ridx_ref[...], (N_TOK, E_LOC))
    le_ids = my * E_LOC + lax.broadcasted_iota(jnp.int32, (N_TOK, E_LOC), 1)
    oh4 = idx == le_ids
    row = lax.broadcasted_iota(jnp.int32, (N_TOK, E_LOC), 0)
    cum4 = oh4.astype(jnp.int32)
    sh = 1
    while sh < N_TOK:
        rolled = jnp.roll(cum4, sh, axis=0)
        cum4 = cum4 + jnp.where(row >= sh, rolled, 0)
        sh *= 2
    mask_ref[...] = (oh4 & (cum4 <= CAP)).astype(jnp.bfloat16)

    xbf_ref[...] = x_ref[...].astype(jnp.bfloat16)
    wbf_ref[...] = w_ref[...].astype(jnp.bfloat16)

    def block(t):
        xb = xbf_ref[pl.ds(t * M_OUT, M_OUT), :]
        mb = mask_ref[pl.ds(t * M_OUT, M_OUT), :]
        acc = jnp.dot(
            xb * mb[:, 0:1], wbf_ref[0], preferred_element_type=jnp.float32
        )
        for le in range(1, E_LOC):
            acc += jnp.dot(
                xb * mb[:, le:le + 1], wbf_ref[le],
                preferred_element_type=jnp.float32,
            )
        return acc

    rdmas = []
    for j in range(1, N_DEV):
        t = (my + j) % N_DEV
        dest_ref[j - 1] = block(t).astype(jnp.bfloat16)
        rdma = pltpu.make_async_remote_copy(
            src_ref=dest_ref.at[j - 1],
            dst_ref=recv_ref.at[3 - j],
            send_sem=send_sems.at[j - 1],
            recv_sem=recv_sems.at[3 - j],
            device_id=(t,),
            device_id_type=pl.DeviceIdType.MESH,
        )
        rdma.start()
        rdmas.append(rdma)

    out_ref[...] = block(my)

    for j in (1, 3, 2):
        rdmas[j - 1].wait_recv()
        out_ref[...] += recv_ref[3 - j].astype(jnp.float32)

    for rdma in rdmas:
        rdma.wait_send()


def kernel(x, router_W, route_idx, expert_W):
    del router_W
    return pl.pallas_call(
        _body,
        out_shape=jax.ShapeDtypeStruct((M_OUT, H), jnp.float32),
        in_specs=[
            pl.BlockSpec(memory_space=pltpu.VMEM),
            pl.BlockSpec(memory_space=pltpu.VMEM),
            pl.BlockSpec(memory_space=pltpu.VMEM),
        ],
        out_specs=pl.BlockSpec(memory_space=pltpu.VMEM),
        scratch_shapes=[
            pltpu.VMEM((N_TOK, D_MODEL), jnp.bfloat16),
            pltpu.VMEM((E_LOC, D_MODEL, H), jnp.bfloat16),
            pltpu.VMEM((N_TOK, E_LOC), jnp.bfloat16),
            pltpu.VMEM((N_DEV - 1, M_OUT, H), jnp.bfloat16),
            pltpu.VMEM((N_DEV - 1, M_OUT, H), jnp.bfloat16),
            pltpu.SemaphoreType.DMA((N_DEV - 1,)),
            pltpu.SemaphoreType.DMA((N_DEV - 1,)),
        ],
        compiler_params=pltpu.CompilerParams(collective_id=0),
    )(x, expert_W, route_idx.astype(jnp.int32))


# device time: 41169 ns/iter; 1.0668x vs baseline; 1.0161x over previous
import jax
import jax.numpy as jnp
from jax import lax
from jax.experimental import pallas as pl
from jax.experimental.pallas import tpu as pltpu

N_DEV = 4
N_TOK = 2048
D_MODEL = 512
H = 1024
N_EXP = 16
E_LOC = 4
CAP = 102
M_OUT = N_TOK // N_DEV


def _body(x_ref, w_ref, ridx_ref, out_ref, mask_ref,
          dest_ref, recv_ref, send_sems, recv_sems):
    my = lax.axis_index("i")

    bar = pltpu.get_barrier_semaphore()
    for k in range(1, N_DEV):
        pl.semaphore_signal(
            bar, inc=1,
            device_id=((my + k) % N_DEV,),
            device_id_type=pl.DeviceIdType.MESH,
        )

    idx = jnp.broadcast_to(ridx_ref[...], (N_TOK, E_LOC))
    le_ids = my * E_LOC + lax.broadcasted_iota(jnp.int32, (N_TOK, E_LOC), 1)
    oh4 = idx == le_ids
    row = lax.broadcasted_iota(jnp.int32, (N_TOK, E_LOC), 0)
    cum4 = oh4.astype(jnp.int32)
    sh = 1
    while sh < N_TOK:
        rolled = jnp.roll(cum4, sh, axis=0)
        cum4 = cum4 + jnp.where(row >= sh, rolled, 0)
        sh *= 2
    mask_ref[...] = (oh4 & (cum4 <= CAP)).astype(jnp.float32)

    def block(t):
        xb = x_ref[pl.ds(t * M_OUT, M_OUT), :]
        mb = mask_ref[pl.ds(t * M_OUT, M_OUT), :]
        acc = jnp.dot(
            xb * mb[:, 0:1], w_ref[0], preferred_element_type=jnp.float32
        )
        for le in range(1, E_LOC):
            acc += jnp.dot(
                xb * mb[:, le:le + 1], w_ref[le],
                preferred_element_type=jnp.float32,
            )
        return acc

    rdmas = []
    for j in range(1, N_DEV):
        t = (my + j) % N_DEV
        dest_ref[j - 1] = block(t).astype(jnp.bfloat16)
        if j == 1:
            pl.semaphore_wait(bar, N_DEV - 1)
        rdma = pltpu.make_async_remote_copy(
            src_ref=dest_ref.at[j - 1],
            dst_ref=recv_ref.at[3 - j],
            send_sem=send_sems.at[j - 1],
            recv_sem=recv_sems.at[3 - j],
            device_id=(t,),
            device_id_type=pl.DeviceIdType.MESH,
        )
        rdma.start()
        rdmas.append(rdma)

    out_ref[...] = block(my)

    for j in (1, 3, 2):
        rdmas[j - 1].wait_recv()
        out_ref[...] += recv_ref[3 - j].astype(jnp.float32)

    for rdma in rdmas:
        rdma.wait_send()


def kernel(x, router_W, route_idx, expert_W):
    del router_W
    return pl.pallas_call(
        _body,
        out_shape=jax.ShapeDtypeStruct((M_OUT, H), jnp.float32),
        in_specs=[
            pl.BlockSpec(memory_space=pltpu.VMEM),
            pl.BlockSpec(memory_space=pltpu.VMEM),
            pl.BlockSpec(memory_space=pltpu.VMEM),
        ],
        out_specs=pl.BlockSpec(memory_space=pltpu.VMEM),
        scratch_shapes=[
            pltpu.VMEM((N_TOK, E_LOC), jnp.float32),
            pltpu.VMEM((N_DEV - 1, M_OUT, H), jnp.bfloat16),
            pltpu.VMEM((N_DEV - 1, M_OUT, H), jnp.bfloat16),
            pltpu.SemaphoreType.DMA((N_DEV - 1,)),
            pltpu.SemaphoreType.DMA((N_DEV - 1,)),
        ],
        compiler_params=pltpu.CompilerParams(collective_id=0),
    )(x, expert_W, route_idx.astype(jnp.int32))
